# baseline (speedup 1.0000x reference)
import functools

import jax
import jax.numpy as jnp
from jax import lax
from jax.experimental import pallas as pl
from jax.experimental.pallas import tpu as pltpu
from jax.experimental.pallas import tpu_sc as plsc

INV_SQRT_2 = 1.0 / 2.0 ** 0.5
KMAX = 8

N_EDGES = 160000
N_INTM = 480000
N_QUAD = 960000
NSLOT = N_EDGES * KMAX
NW = 32
SLOTS_PER = NSLOT // NW
SROWS = 625
SCOLS = 64
QCH = 100
QROWS = 600
PAD = 3200
BIGKEY = 0x7FF00000

_SDS = jax.ShapeDtypeStruct

_SC_PARAMS = pltpu.CompilerParams(needs_layout_passes=False,
                                  use_tc_tiling_on_sc=False)


def _edge_dense_body(m_ref, rbf_ref, wdb_ref, wrbf_ref, wdown_ref, out_ref):
    x = jax.nn.silu(jnp.dot(m_ref[...], wdb_ref[...],
                            preferred_element_type=jnp.float32))
    x = x * jnp.dot(rbf_ref[...], wrbf_ref[...],
                    preferred_element_type=jnp.float32)
    out_ref[...] = jax.nn.silu(jnp.dot(x, wdown_ref[...],
                                       preferred_element_type=jnp.float32))


def _edge_dense(m, rbf, W_db, W_rbf, W_down, block=640):
    n = m.shape[0]
    return pl.pallas_call(
        _edge_dense_body,
        grid=(n // block,),
        in_specs=[
            pl.BlockSpec((block, m.shape[1]), lambda i: (i, 0)),
            pl.BlockSpec((block, rbf.shape[1]), lambda i: (i, 0)),
            pl.BlockSpec(W_db.shape, lambda i: (0, 0)),
            pl.BlockSpec(W_rbf.shape, lambda i: (0, 0)),
            pl.BlockSpec(W_down.shape, lambda i: (0, 0)),
        ],
        out_specs=pl.BlockSpec((block, W_down.shape[1]), lambda i: (i, 0)),
        out_shape=_SDS((n, W_down.shape[1]), jnp.float32),
    )(m, rbf, W_db, W_rbf, W_down)


def _matmul_pad_body(nblk, a_ref, w_ref, out_ref):
    i = pl.program_id(0)

    @pl.when(i < nblk)
    def _():
        out_ref[...] = jnp.dot(a_ref[...], w_ref[...],
                               preferred_element_type=jnp.float32)

    @pl.when(i >= nblk)
    def _():
        out_ref[...] = jnp.zeros_like(out_ref)


def _matmul_pad(a, w, pad_rows, block=1600):
    n = a.shape[0]
    nblk = n // block
    gpad = pad_rows // block
    return pl.pallas_call(
        functools.partial(_matmul_pad_body, nblk),
        grid=(nblk + gpad,),
        in_specs=[
            pl.BlockSpec((block, a.shape[1]),
                         lambda i: (jnp.minimum(i, nblk - 1), 0)),
            pl.BlockSpec(w.shape, lambda i: (0, 0)),
        ],
        out_specs=pl.BlockSpec((block, w.shape[1]), lambda i: (i, 0)),
        out_shape=_SDS((n + pad_rows, w.shape[1]), jnp.float32),
    )(a, w)


def _bilinear_body(sph_ref, rbfw1_ref, m2x_ref, m2c_ref, wbil_ref, out_ref):
    e = sph_ref.shape[0]
    sph = sph_ref[...].reshape(e, 8, KMAX)
    rbT = rbfw1_ref[...].reshape(e, 8, 32)
    m2 = (m2x_ref[...] * m2c_ref[...]).reshape(e, KMAX, 32)
    sum_k = lax.dot_general(
        sph, m2, (((2,), (1,)), ((0,), (0,))),
        preferred_element_type=jnp.float32)
    t = lax.dot_general(
        rbT, sum_k, (((1,), (1,)), ((0,), (0,))),
        preferred_element_type=jnp.float32)
    t2 = t.reshape(e, 32 * 32)
    out_ref[...] = jnp.dot(t2, wbil_ref[...],
                           preferred_element_type=jnp.float32)


def _bilinear(sph2, rbfw1T, m2x, m2c, W2, block=320):
    n = sph2.shape[0]
    return pl.pallas_call(
        _bilinear_body,
        grid=(n // block,),
        in_specs=[
            pl.BlockSpec((block, 64), lambda i: (i, 0)),
            pl.BlockSpec((block * KMAX, 32), lambda i: (i, 0)),
            pl.BlockSpec((block * KMAX, 32), lambda i: (i, 0)),
            pl.BlockSpec((block * KMAX, 32), lambda i: (i, 0)),
            pl.BlockSpec(W2.shape, lambda i: (0, 0)),
        ],
        out_specs=pl.BlockSpec((block, 32), lambda i: (i, 0)),
        out_shape=_SDS((n, 32), jnp.float32),
    )(sph2, rbfw1T, m2x, m2c, W2)


def _up_body(x_ref, xsw_ref, wca_ref, wac_ref, out_ref):
    x_ca = jax.nn.silu(jnp.dot(x_ref[...], wca_ref[...],
                               preferred_element_type=jnp.float32))
    x_ac = jax.nn.silu(jnp.dot(xsw_ref[...], wac_ref[...],
                               preferred_element_type=jnp.float32))
    out_ref[...] = (x_ca + x_ac) * INV_SQRT_2


def _up(x, x_sw, W_up_ca, W_up_ac, block=640):
    n = x.shape[0]
    return pl.pallas_call(
        _up_body,
        grid=(n // block,),
        in_specs=[
            pl.BlockSpec((block, 32), lambda i: (i, 0)),
            pl.BlockSpec((block, 32), lambda i: (i, 0)),
            pl.BlockSpec(W_up_ca.shape, lambda i: (0, 0)),
            pl.BlockSpec(W_up_ac.shape, lambda i: (0, 0)),
        ],
        out_specs=pl.BlockSpec((block, 128), lambda i: (i, 0)),
        out_shape=_SDS((n, 128), jnp.float32),
    )(x, x_sw, W_up_ca, W_up_ac)


def _lane_shift_up(x, iota):
    idx = jnp.minimum(iota + 1, 15)
    return lax.gather(
        x, idx[:, None],
        lax.GatherDimensionNumbers(offset_dims=(), collapsed_slice_dims=(0,),
                                   start_index_map=(0,)),
        (1,), mode=lax.GatherScatterMode.PROMISE_IN_BOUNDS)


def _sc_p1_body(ca_hbm, kid_hbm, win_hbm, wbuf, cabuf, kbuf):
    cid = lax.axis_index("c")
    sid = lax.axis_index("s")
    wid = sid * 2 + cid
    slot_base = wid * SLOTS_PER
    iota = lax.iota(jnp.int32, 16)
    neg1 = jnp.full((16,), -1, jnp.int32)
    bigv = BIGKEY + iota
    last_lane = iota == 15

    @pl.loop(0, SROWS)
    def _init(r):
        for t in range(SCOLS // 16):
            wbuf[r, pl.ds(t * 16, 16)] = neg1

    @pl.loop(0, QCH)
    def _p1(c):
        pltpu.sync_copy(ca_hbm.at[c], cabuf)
        pltpu.sync_copy(kid_hbm.at[c], kbuf)

        @pl.loop(0, QROWS, unroll=2)
        def _p1j(j):
            ca_v = cabuf[j, :]
            k_v = kbuf[j, :]
            loc = ca_v * 8 + k_v - slot_base
            owned = (loc >= 0) & (loc < SLOTS_PER)

            @pl.when(jnp.any(owned))
            def _():
                key = jnp.where(owned, loc * 16 + iota, bigv)
                q_v = (c * (QROWS * 16) + j * 16) + iota
                ks, qs = plsc.sort_key_val(key, q_v)
                sloc = lax.shift_right_logical(ks, 4)
                valid = ks < BIGKEY
                nxt = _lane_shift_up(sloc, iota)
                keep = valid & ((sloc != nxt) | last_lane)
                slc = jnp.minimum(sloc, SLOTS_PER - 1)
                row = lax.shift_right_logical(slc, 6)
                col = slc & (SCOLS - 1)
                plsc.store_scatter(wbuf, [row, col], qs, mask=keep)

    pltpu.sync_copy(wbuf, win_hbm.at[wid])


def _sc_phase1(ca3, kid3):
    mesh = plsc.VectorSubcoreMesh(core_axis_name="c", subcore_axis_name="s")
    f = pl.kernel(
        _sc_p1_body,
        out_type=_SDS((NW, SROWS, SCOLS), jnp.int32),
        mesh=mesh,
        compiler_params=_SC_PARAMS,
        scratch_types=[
            pltpu.VMEM((SROWS, SCOLS), jnp.int32),
            pltpu.VMEM((QROWS, 16), jnp.int32),
            pltpu.VMEM((QROWS, 16), jnp.int32),
        ],
    )
    return f(ca3, kid3)


def _sc_p2_body(win_hbm, abd_hbm, db_hbm, ccoef_hbm, xedge_hbm,
                m2x_hbm, m2c_hbm, wbuf, ibuf, crow, xrow, semg, sems):
    cid = lax.axis_index("c")
    sid = lax.axis_index("s")
    wid = sid * 2 + cid
    slot_base = wid * SLOTS_PER
    iota = lax.iota(jnp.int32, 16)

    pltpu.sync_copy(win_hbm.at[wid], wbuf)

    @pl.loop(0, SROWS)
    def _p2a(r):
        for t in range(SCOLS // 16):
            w = wbuf[r, pl.ds(t * 16, 16)]
            ls = r * SCOLS + t * 16 + iota
            dummy = N_QUAD + (ls & 2047)
            wbuf[r, pl.ds(t * 16, 16)] = jnp.where(w >= 0, w, dummy)

    @pl.loop(0, SROWS // 25)
    def _p2b(g):
        cps = []
        for b in range(25):
            c = g * 25 + b
            cps.append(pltpu.async_copy(abd_hbm.at[wbuf.at[c]],
                                        ibuf.at[c], semg))
        for cp in cps:
            cp.wait()

    @pl.loop(0, SROWS // 25)
    def _p2c(g):
        cps = []
        for b in range(25):
            c = g * 25 + b
            cps.append(pltpu.async_copy(db_hbm.at[ibuf.at[c]],
                                        wbuf.at[c], semg))
        for cp in cps:
            cp.wait()

    @pl.loop(0, SROWS // 5)
    def _p2d(g):
        gcps = []
        for b in range(5):
            c = g * 5 + b
            gcps.append(pltpu.async_copy(ccoef_hbm.at[ibuf.at[c]],
                                         crow.at[b], semg))
            gcps.append(pltpu.async_copy(xedge_hbm.at[wbuf.at[c]],
                                         xrow.at[b], semg))
        scps = []
        for b in range(5):
            c = g * 5 + b
            gcps[2 * b].wait()
            gcps[2 * b + 1].wait()
            base = slot_base + c * SCOLS
            scps.append(pltpu.async_copy(crow.at[b],
                                         m2c_hbm.at[pl.ds(base, SCOLS)], sems))
            scps.append(pltpu.async_copy(xrow.at[b],
                                         m2x_hbm.at[pl.ds(base, SCOLS)], sems))
        for cp in scps:
            cp.wait()


def _sc_phase2(win, abd_pad, db_pad, ccoef_pad, x_edge):
    mesh = plsc.VectorSubcoreMesh(core_axis_name="c", subcore_axis_name="s")
    f = pl.kernel(
        _sc_p2_body,
        out_type=(_SDS((NSLOT, 32), jnp.float32),
                  _SDS((NSLOT, 32), jnp.float32)),
        mesh=mesh,
        compiler_params=_SC_PARAMS,
        scratch_types=[
            pltpu.VMEM((SROWS, SCOLS), jnp.int32),
            pltpu.VMEM((SROWS, SCOLS), jnp.int32),
            pltpu.VMEM((5, SCOLS, 32), jnp.float32),
            pltpu.VMEM((5, SCOLS, 32), jnp.float32),
            pltpu.SemaphoreType.DMA,
            pltpu.SemaphoreType.DMA,
        ],
    )
    return f(win, abd_pad, db_pad, ccoef_pad, x_edge)


def _sc_swap_body(x_hbm, idsw_hbm, out_hbm, idxbuf, rbuf, semg, sems):
    cid = lax.axis_index("c")
    sid = lax.axis_index("s")
    wid = sid * 2 + cid
    pltpu.sync_copy(idsw_hbm.at[wid], idxbuf)

    @pl.loop(0, 25)
    def _g(g):
        gcps = []
        for b in range(5):
            c = g * 5 + b
            gcps.append(pltpu.async_copy(x_hbm.at[idxbuf.at[c]],
                                         rbuf.at[b], semg))
        scps = []
        for b in range(5):
            c = g * 5 + b
            gcps[b].wait()
            base = wid * 5000 + c * 40
            scps.append(pltpu.async_copy(rbuf.at[b],
                                         out_hbm.at[pl.ds(base, 40)], sems))
        for cp in scps:
            cp.wait()


def _sc_swap(x, id_swap):
    mesh = plsc.VectorSubcoreMesh(core_axis_name="c", subcore_axis_name="s")
    f = pl.kernel(
        _sc_swap_body,
        out_type=_SDS((N_EDGES, 32), jnp.float32),
        mesh=mesh,
        compiler_params=_SC_PARAMS,
        scratch_types=[
            pltpu.VMEM((125, 40), jnp.int32),
            pltpu.VMEM((5, 40, 32), jnp.float32),
            pltpu.SemaphoreType.DMA,
            pltpu.SemaphoreType.DMA,
        ],
    )
    return f(x, id_swap.reshape(NW, 125, 40))


def kernel(m, rbf, cbf, sbf_rbfW1, sbf_sph, Kidx4, id_swap, id4_reduce_ca,
           id4_expand_intm_db, id4_expand_abd,
           W_db, W_rbf, W_cbf, W_down, W_bil, W_up_ca, W_up_ac):
    ca3 = id4_reduce_ca.reshape(QCH, QROWS, 16)
    kid3 = Kidx4.reshape(QCH, QROWS, 16)
    win = _sc_phase1(ca3, kid3)

    x_edge = _edge_dense(m, rbf, W_db, W_rbf, W_down)
    ccoef_pad = _matmul_pad(cbf, W_cbf, PAD)

    abd_pad = jnp.concatenate(
        [id4_expand_abd, N_INTM + jnp.arange(PAD, dtype=jnp.int32)])
    db_pad = jnp.concatenate(
        [id4_expand_intm_db,
         (jnp.arange(PAD, dtype=jnp.int32) * 19) % N_EDGES])

    m2x, m2c = _sc_phase2(win, abd_pad, db_pad, ccoef_pad, x_edge)

    rbfw1T = sbf_rbfW1.transpose(0, 2, 1).reshape(NSLOT, 32)
    W2 = W_bil.transpose(1, 0, 2).reshape(32 * 32, 32)
    x = _bilinear(sbf_sph.reshape(N_EDGES, 64),
                  rbfw1T, m2x, m2c, W2)
    x_sw = _sc_swap(x, id_swap)
    return _up(x, x_sw, W_up_ca, W_up_ac)


# speedup vs baseline: 1.2624x; 1.2624x over previous
import functools

import jax
import jax.numpy as jnp
from jax import lax
from jax.experimental import pallas as pl
from jax.experimental.pallas import tpu as pltpu
from jax.experimental.pallas import tpu_sc as plsc

INV_SQRT_2 = 1.0 / 2.0 ** 0.5
KMAX = 8

N_EDGES = 160000
N_INTM = 480000
N_QUAD = 960000
NSLOT = N_EDGES * KMAX
NW = 32
SLOTS_PER = NSLOT // NW
SROWS = 625
SCOLS = 64
QCH = 100
QROWS = 600
PAD = 3200
BIGKEY = 0x7FF00000

_SDS = jax.ShapeDtypeStruct

_SC_PARAMS = pltpu.CompilerParams(needs_layout_passes=False,
                                  use_tc_tiling_on_sc=False)


def _edge_dense_body(m_ref, rbf_ref, wdb_ref, wrbf_ref, wdown_ref, out_ref):
    x = jax.nn.silu(jnp.dot(m_ref[...], wdb_ref[...],
                            preferred_element_type=jnp.float32))
    x = x * jnp.dot(rbf_ref[...], wrbf_ref[...],
                    preferred_element_type=jnp.float32)
    out_ref[...] = jax.nn.silu(jnp.dot(x, wdown_ref[...],
                                       preferred_element_type=jnp.float32))


def _edge_dense(m, rbf, W_db, W_rbf, W_down, block=640):
    n = m.shape[0]
    return pl.pallas_call(
        _edge_dense_body,
        grid=(n // block,),
        in_specs=[
            pl.BlockSpec((block, m.shape[1]), lambda i: (i, 0)),
            pl.BlockSpec((block, rbf.shape[1]), lambda i: (i, 0)),
            pl.BlockSpec(W_db.shape, lambda i: (0, 0)),
            pl.BlockSpec(W_rbf.shape, lambda i: (0, 0)),
            pl.BlockSpec(W_down.shape, lambda i: (0, 0)),
        ],
        out_specs=pl.BlockSpec((block, W_down.shape[1]), lambda i: (i, 0)),
        out_shape=_SDS((n, W_down.shape[1]), jnp.float32),
    )(m, rbf, W_db, W_rbf, W_down)


def _matmul_pad_body(nblk, a_ref, w_ref, out_ref):
    i = pl.program_id(0)

    @pl.when(i < nblk)
    def _():
        out_ref[...] = jnp.dot(a_ref[...], w_ref[...],
                               preferred_element_type=jnp.float32)

    @pl.when(i >= nblk)
    def _():
        out_ref[...] = jnp.zeros_like(out_ref)


def _matmul_pad(a, w, pad_rows, block=1600):
    n = a.shape[0]
    nblk = n // block
    gpad = pad_rows // block
    return pl.pallas_call(
        functools.partial(_matmul_pad_body, nblk),
        grid=(nblk + gpad,),
        in_specs=[
            pl.BlockSpec((block, a.shape[1]),
                         lambda i: (jnp.minimum(i, nblk - 1), 0)),
            pl.BlockSpec(w.shape, lambda i: (0, 0)),
        ],
        out_specs=pl.BlockSpec((block, w.shape[1]), lambda i: (i, 0)),
        out_shape=_SDS((n + pad_rows, w.shape[1]), jnp.float32),
    )(a, w)


def _bilinear_body(sph_ref, rbfw1_ref, m2_ref, wbil_ref, out_ref):
    e = sph_ref.shape[0]
    sph = sph_ref[...].reshape(e, 8, KMAX)
    rbT = rbfw1_ref[...].reshape(e, 8, 32)
    m2 = m2_ref[...].reshape(e, KMAX, 32)
    sum_k = lax.dot_general(
        sph, m2, (((2,), (1,)), ((0,), (0,))),
        preferred_element_type=jnp.float32)
    t = lax.dot_general(
        rbT, sum_k, (((1,), (1,)), ((0,), (0,))),
        preferred_element_type=jnp.float32)
    t2 = t.reshape(e, 32 * 32)
    out_ref[...] = jnp.dot(t2, wbil_ref[...],
                           preferred_element_type=jnp.float32)


def _bilinear(sph2, rbfw1T, m2, W2, block=320):
    n = sph2.shape[0]
    return pl.pallas_call(
        _bilinear_body,
        grid=(n // block,),
        in_specs=[
            pl.BlockSpec((block, 64), lambda i: (i, 0)),
            pl.BlockSpec((block * KMAX, 32), lambda i: (i, 0)),
            pl.BlockSpec((block * KMAX, 32), lambda i: (i, 0)),
            pl.BlockSpec(W2.shape, lambda i: (0, 0)),
        ],
        out_specs=pl.BlockSpec((block, 32), lambda i: (i, 0)),
        out_shape=_SDS((n, 32), jnp.float32),
    )(sph2, rbfw1T, m2, W2)


def _up_body(x_ref, xsw_ref, wca_ref, wac_ref, out_ref):
    x_ca = jax.nn.silu(jnp.dot(x_ref[...], wca_ref[...],
                               preferred_element_type=jnp.float32))
    x_ac = jax.nn.silu(jnp.dot(xsw_ref[...], wac_ref[...],
                               preferred_element_type=jnp.float32))
    out_ref[...] = (x_ca + x_ac) * INV_SQRT_2


def _up(x, x_sw, W_up_ca, W_up_ac, block=640):
    n = x.shape[0]
    return pl.pallas_call(
        _up_body,
        grid=(n // block,),
        in_specs=[
            pl.BlockSpec((block, 32), lambda i: (i, 0)),
            pl.BlockSpec((block, 32), lambda i: (i, 0)),
            pl.BlockSpec(W_up_ca.shape, lambda i: (0, 0)),
            pl.BlockSpec(W_up_ac.shape, lambda i: (0, 0)),
        ],
        out_specs=pl.BlockSpec((block, 128), lambda i: (i, 0)),
        out_shape=_SDS((n, 128), jnp.float32),
    )(x, x_sw, W_up_ca, W_up_ac)


def _lane_shift_up(x, iota):
    idx = jnp.minimum(iota + 1, 15)
    return lax.gather(
        x, idx[:, None],
        lax.GatherDimensionNumbers(offset_dims=(), collapsed_slice_dims=(0,),
                                   start_index_map=(0,)),
        (1,), mode=lax.GatherScatterMode.PROMISE_IN_BOUNDS)


def _sc_p1_body(ca_hbm, kid_hbm, win_hbm, wbuf, cabuf, kbuf):
    cid = lax.axis_index("c")
    sid = lax.axis_index("s")
    wid = sid * 2 + cid
    slot_base = wid * SLOTS_PER
    iota = lax.iota(jnp.int32, 16)
    neg1 = jnp.full((16,), -1, jnp.int32)
    bigv = BIGKEY + iota
    last_lane = iota == 15

    @pl.loop(0, SROWS)
    def _init(r):
        for t in range(SCOLS // 16):
            wbuf[r, pl.ds(t * 16, 16)] = neg1

    @pl.loop(0, QCH)
    def _p1(c):
        pltpu.sync_copy(ca_hbm.at[c], cabuf)
        pltpu.sync_copy(kid_hbm.at[c], kbuf)

        @pl.loop(0, QROWS, unroll=8)
        def _p1j(j):
            ca_v = cabuf[j, :]
            k_v = kbuf[j, :]
            loc = ca_v * 8 + k_v - slot_base
            owned = (loc >= 0) & (loc < SLOTS_PER)
            key = jnp.where(owned, loc * 16 + iota, bigv)
            q_v = (c * (QROWS * 16) + j * 16) + iota
            ks, qs = plsc.sort_key_val(key, q_v)
            sloc = lax.shift_right_logical(ks, 4)
            valid = ks < BIGKEY
            nxt = _lane_shift_up(sloc, iota)
            keep = valid & ((sloc != nxt) | last_lane)
            slc = jnp.minimum(sloc, SLOTS_PER - 1)
            row = lax.shift_right_logical(slc, 6)
            col = slc & (SCOLS - 1)
            plsc.store_scatter(wbuf, [row, col], qs, mask=keep)

    pltpu.sync_copy(wbuf, win_hbm.at[wid])


def _sc_phase1(ca3, kid3):
    mesh = plsc.VectorSubcoreMesh(core_axis_name="c", subcore_axis_name="s")
    f = pl.kernel(
        _sc_p1_body,
        out_type=_SDS((NW, SROWS, SCOLS), jnp.int32),
        mesh=mesh,
        compiler_params=_SC_PARAMS,
        scratch_types=[
            pltpu.VMEM((SROWS, SCOLS), jnp.int32),
            pltpu.VMEM((QROWS, 16), jnp.int32),
            pltpu.VMEM((QROWS, 16), jnp.int32),
        ],
    )
    return f(ca3, kid3)


def _sc_p2_body(win_hbm, abd_hbm, db_hbm, ccoef_hbm, xedge_hbm,
                m2x_hbm, wbuf, ibuf, crow, xrow, semg, sems):
    cid = lax.axis_index("c")
    sid = lax.axis_index("s")
    wid = sid * 2 + cid
    slot_base = wid * SLOTS_PER
    iota = lax.iota(jnp.int32, 16)

    pltpu.sync_copy(win_hbm.at[wid], wbuf)

    @pl.loop(0, SROWS)
    def _p2a(r):
        for t in range(SCOLS // 16):
            w = wbuf[r, pl.ds(t * 16, 16)]
            ls = r * SCOLS + t * 16 + iota
            dummy = N_QUAD + (ls & 2047)
            wbuf[r, pl.ds(t * 16, 16)] = jnp.where(w >= 0, w, dummy)

    @pl.loop(0, SROWS // 25)
    def _p2b(g):
        cps = []
        for b in range(25):
            c = g * 25 + b
            cps.append(pltpu.async_copy(abd_hbm.at[wbuf.at[c]],
                                        ibuf.at[c], semg))
        for cp in cps:
            cp.wait()

    @pl.loop(0, SROWS // 25)
    def _p2c(g):
        cps = []
        for b in range(25):
            c = g * 25 + b
            cps.append(pltpu.async_copy(db_hbm.at[ibuf.at[c]],
                                        wbuf.at[c], semg))
        for cp in cps:
            cp.wait()

    @pl.loop(0, SROWS // 5)
    def _p2d(g):
        gcps = []
        for b in range(5):
            c = g * 5 + b
            gcps.append(pltpu.async_copy(ccoef_hbm.at[ibuf.at[c]],
                                         crow.at[b], semg))
            gcps.append(pltpu.async_copy(xedge_hbm.at[wbuf.at[c]],
                                         xrow.at[b], semg))
        scps = []
        for b in range(5):
            c = g * 5 + b
            gcps[2 * b].wait()
            gcps[2 * b + 1].wait()
            for r in range(SCOLS):
                for t in range(2):
                    sl = pl.ds(t * 16, 16)
                    crow[b, r, sl] = crow[b, r, sl] * xrow[b, r, sl]
            base = slot_base + c * SCOLS
            scps.append(pltpu.async_copy(crow.at[b],
                                         m2x_hbm.at[pl.ds(base, SCOLS)], sems))
        for cp in scps:
            cp.wait()


def _sc_phase2(win, abd_pad, db_pad, ccoef_pad, x_edge):
    mesh = plsc.VectorSubcoreMesh(core_axis_name="c", subcore_axis_name="s")
    f = pl.kernel(
        _sc_p2_body,
        out_type=_SDS((NSLOT, 32), jnp.float32),
        mesh=mesh,
        compiler_params=_SC_PARAMS,
        scratch_types=[
            pltpu.VMEM((SROWS, SCOLS), jnp.int32),
            pltpu.VMEM((SROWS, SCOLS), jnp.int32),
            pltpu.VMEM((5, SCOLS, 32), jnp.float32),
            pltpu.VMEM((5, SCOLS, 32), jnp.float32),
            pltpu.SemaphoreType.DMA,
            pltpu.SemaphoreType.DMA,
        ],
    )
    return f(win, abd_pad, db_pad, ccoef_pad, x_edge)


def _sc_swap_body(x_hbm, idsw_hbm, out_hbm, idxbuf, rbuf, semg, sems):
    cid = lax.axis_index("c")
    sid = lax.axis_index("s")
    wid = sid * 2 + cid
    pltpu.sync_copy(idsw_hbm.at[wid], idxbuf)

    @pl.loop(0, 25)
    def _g(g):
        gcps = []
        for b in range(5):
            c = g * 5 + b
            gcps.append(pltpu.async_copy(x_hbm.at[idxbuf.at[c]],
                                         rbuf.at[b], semg))
        scps = []
        for b in range(5):
            c = g * 5 + b
            gcps[b].wait()
            base = wid * 5000 + c * 40
            scps.append(pltpu.async_copy(rbuf.at[b],
                                         out_hbm.at[pl.ds(base, 40)], sems))
        for cp in scps:
            cp.wait()


def _sc_swap(x, id_swap):
    mesh = plsc.VectorSubcoreMesh(core_axis_name="c", subcore_axis_name="s")
    f = pl.kernel(
        _sc_swap_body,
        out_type=_SDS((N_EDGES, 32), jnp.float32),
        mesh=mesh,
        compiler_params=_SC_PARAMS,
        scratch_types=[
            pltpu.VMEM((125, 40), jnp.int32),
            pltpu.VMEM((5, 40, 32), jnp.float32),
            pltpu.SemaphoreType.DMA,
            pltpu.SemaphoreType.DMA,
        ],
    )
    return f(x, id_swap.reshape(NW, 125, 40))


def kernel(m, rbf, cbf, sbf_rbfW1, sbf_sph, Kidx4, id_swap, id4_reduce_ca,
           id4_expand_intm_db, id4_expand_abd,
           W_db, W_rbf, W_cbf, W_down, W_bil, W_up_ca, W_up_ac):
    ca3 = id4_reduce_ca.reshape(QCH, QROWS, 16)
    kid3 = Kidx4.reshape(QCH, QROWS, 16)
    win = _sc_phase1(ca3, kid3)

    x_edge = _edge_dense(m, rbf, W_db, W_rbf, W_down)
    ccoef_pad = _matmul_pad(cbf, W_cbf, PAD)

    abd_pad = jnp.concatenate(
        [id4_expand_abd, N_INTM + jnp.arange(PAD, dtype=jnp.int32)])
    db_pad = jnp.concatenate(
        [id4_expand_intm_db,
         (jnp.arange(PAD, dtype=jnp.int32) * 19) % N_EDGES])

    m2 = _sc_phase2(win, abd_pad, db_pad, ccoef_pad, x_edge)

    rbfw1T = sbf_rbfW1.transpose(0, 2, 1).reshape(NSLOT, 32)
    W2 = W_bil.transpose(1, 0, 2).reshape(32 * 32, 32)
    x = _bilinear(sbf_sph.reshape(N_EDGES, 64),
                  rbfw1T, m2, W2)
    x_sw = _sc_swap(x, id_swap)
    return _up(x, x_sw, W_up_ca, W_up_ac)


# speedup vs baseline: 1.2895x; 1.0215x over previous
import functools

import jax
import jax.numpy as jnp
from jax import lax
from jax.experimental import pallas as pl
from jax.experimental.pallas import tpu as pltpu
from jax.experimental.pallas import tpu_sc as plsc

INV_SQRT_2 = 1.0 / 2.0 ** 0.5
KMAX = 8

N_EDGES = 160000
N_INTM = 480000
N_QUAD = 960000
NSLOT = N_EDGES * KMAX
NW = 32
SLOTS_PER = NSLOT // NW
SROWS = 625
SCOLS = 64
QCH = 100
QROWS = 600
PAD = 3200
BIGKEY = 0x7FF00000

_SDS = jax.ShapeDtypeStruct

_SC_PARAMS = pltpu.CompilerParams(needs_layout_passes=False,
                                  use_tc_tiling_on_sc=False)


def _edge_dense_body(m_ref, rbf_ref, wdb_ref, wrbf_ref, wdown_ref, out_ref):
    x = jax.nn.silu(jnp.dot(m_ref[...], wdb_ref[...],
                            preferred_element_type=jnp.float32))
    x = x * jnp.dot(rbf_ref[...], wrbf_ref[...],
                    preferred_element_type=jnp.float32)
    out_ref[...] = jax.nn.silu(jnp.dot(x, wdown_ref[...],
                                       preferred_element_type=jnp.float32))


def _edge_dense(m, rbf, W_db, W_rbf, W_down, block=640):
    n = m.shape[0]
    return pl.pallas_call(
        _edge_dense_body,
        grid=(n // block,),
        in_specs=[
            pl.BlockSpec((block, m.shape[1]), lambda i: (i, 0)),
            pl.BlockSpec((block, rbf.shape[1]), lambda i: (i, 0)),
            pl.BlockSpec(W_db.shape, lambda i: (0, 0)),
            pl.BlockSpec(W_rbf.shape, lambda i: (0, 0)),
            pl.BlockSpec(W_down.shape, lambda i: (0, 0)),
        ],
        out_specs=pl.BlockSpec((block, W_down.shape[1]), lambda i: (i, 0)),
        out_shape=_SDS((n, W_down.shape[1]), jnp.float32),
    )(m, rbf, W_db, W_rbf, W_down)


def _matmul_pad_body(nblk, a_ref, w_ref, out_ref):
    i = pl.program_id(0)

    @pl.when(i < nblk)
    def _():
        out_ref[...] = jnp.dot(a_ref[...], w_ref[...],
                               preferred_element_type=jnp.float32)

    @pl.when(i >= nblk)
    def _():
        out_ref[...] = jnp.zeros_like(out_ref)


def _matmul_pad(a, w, pad_rows, block=1600):
    n = a.shape[0]
    nblk = n // block
    gpad = pad_rows // block
    return pl.pallas_call(
        functools.partial(_matmul_pad_body, nblk),
        grid=(nblk + gpad,),
        in_specs=[
            pl.BlockSpec((block, a.shape[1]),
                         lambda i: (jnp.minimum(i, nblk - 1), 0)),
            pl.BlockSpec(w.shape, lambda i: (0, 0)),
        ],
        out_specs=pl.BlockSpec((block, w.shape[1]), lambda i: (i, 0)),
        out_shape=_SDS((n + pad_rows, w.shape[1]), jnp.float32),
    )(a, w)


def _bilinear_body(sph_ref, rbfw1_ref, m2_ref, wbil_ref, out_ref):
    e = sph_ref.shape[0]
    sph = sph_ref[...].reshape(e, 8, KMAX)
    rbT = rbfw1_ref[...].reshape(e, 8, 32)
    m2 = m2_ref[...].reshape(e, KMAX, 32)
    sum_k = lax.dot_general(
        sph, m2, (((2,), (1,)), ((0,), (0,))),
        preferred_element_type=jnp.float32)
    t = lax.dot_general(
        rbT, sum_k, (((1,), (1,)), ((0,), (0,))),
        preferred_element_type=jnp.float32)
    t2 = t.reshape(e, 32 * 32)
    out_ref[...] = jnp.dot(t2, wbil_ref[...],
                           preferred_element_type=jnp.float32)


def _bilinear(sph2, rbfw1T, m2, W2, block=640):
    n = sph2.shape[0]
    return pl.pallas_call(
        _bilinear_body,
        grid=(n // block,),
        in_specs=[
            pl.BlockSpec((block, 64), lambda i: (i, 0)),
            pl.BlockSpec((block * KMAX, 32), lambda i: (i, 0)),
            pl.BlockSpec((block * KMAX, 32), lambda i: (i, 0)),
            pl.BlockSpec(W2.shape, lambda i: (0, 0)),
        ],
        out_specs=pl.BlockSpec((block, 32), lambda i: (i, 0)),
        out_shape=_SDS((n, 32), jnp.float32),
    )(sph2, rbfw1T, m2, W2)


def _up_body(x_ref, xsw_ref, wca_ref, wac_ref, out_ref):
    x_ca = jax.nn.silu(jnp.dot(x_ref[...], wca_ref[...],
                               preferred_element_type=jnp.float32))
    x_ac = jax.nn.silu(jnp.dot(xsw_ref[...], wac_ref[...],
                               preferred_element_type=jnp.float32))
    out_ref[...] = (x_ca + x_ac) * INV_SQRT_2


def _up(x, x_sw, W_up_ca, W_up_ac, block=640):
    n = x.shape[0]
    return pl.pallas_call(
        _up_body,
        grid=(n // block,),
        in_specs=[
            pl.BlockSpec((block, 32), lambda i: (i, 0)),
            pl.BlockSpec((block, 32), lambda i: (i, 0)),
            pl.BlockSpec(W_up_ca.shape, lambda i: (0, 0)),
            pl.BlockSpec(W_up_ac.shape, lambda i: (0, 0)),
        ],
        out_specs=pl.BlockSpec((block, 128), lambda i: (i, 0)),
        out_shape=_SDS((n, 128), jnp.float32),
    )(x, x_sw, W_up_ca, W_up_ac)


def _lane_shift_up(x, iota):
    idx = jnp.minimum(iota + 1, 15)
    return lax.gather(
        x, idx[:, None],
        lax.GatherDimensionNumbers(offset_dims=(), collapsed_slice_dims=(0,),
                                   start_index_map=(0,)),
        (1,), mode=lax.GatherScatterMode.PROMISE_IN_BOUNDS)


def _sc_p1_body(ca_hbm, kid_hbm, win_hbm, wbuf, cabuf, kbuf):
    cid = lax.axis_index("c")
    sid = lax.axis_index("s")
    wid = sid * 2 + cid
    slot_base = wid * SLOTS_PER
    iota = lax.iota(jnp.int32, 16)
    neg1 = jnp.full((16,), -1, jnp.int32)
    bigv = BIGKEY + iota
    last_lane = iota == 15

    @pl.loop(0, SROWS)
    def _init(r):
        for t in range(SCOLS // 16):
            wbuf[r, pl.ds(t * 16, 16)] = neg1

    @pl.loop(0, QCH)
    def _p1(c):
        pltpu.sync_copy(ca_hbm.at[c], cabuf)
        pltpu.sync_copy(kid_hbm.at[c], kbuf)

        @pl.loop(0, QROWS, unroll=8)
        def _p1j(j):
            ca_v = cabuf[j, :]
            k_v = kbuf[j, :]
            loc = ca_v * 8 + k_v - slot_base
            owned = (loc >= 0) & (loc < SLOTS_PER)
            key = jnp.where(owned, loc * 16 + iota, bigv)
            q_v = (c * (QROWS * 16) + j * 16) + iota
            ks, qs = plsc.sort_key_val(key, q_v)
            sloc = lax.shift_right_logical(ks, 4)
            valid = ks < BIGKEY
            nxt = _lane_shift_up(sloc, iota)
            keep = valid & ((sloc != nxt) | last_lane)
            slc = jnp.minimum(sloc, SLOTS_PER - 1)
            row = lax.shift_right_logical(slc, 6)
            col = slc & (SCOLS - 1)
            plsc.store_scatter(wbuf, [row, col], qs, mask=keep)

    pltpu.sync_copy(wbuf, win_hbm.at[wid])


def _sc_phase1(ca3, kid3):
    mesh = plsc.VectorSubcoreMesh(core_axis_name="c", subcore_axis_name="s")
    f = pl.kernel(
        _sc_p1_body,
        out_type=_SDS((NW, SROWS, SCOLS), jnp.int32),
        mesh=mesh,
        compiler_params=_SC_PARAMS,
        scratch_types=[
            pltpu.VMEM((SROWS, SCOLS), jnp.int32),
            pltpu.VMEM((QROWS, 16), jnp.int32),
            pltpu.VMEM((QROWS, 16), jnp.int32),
        ],
    )
    return f(ca3, kid3)


def _sc_p2_body(win_hbm, abd_hbm, db_hbm, ccoef_hbm, xedge_hbm,
                m2x_hbm, wbuf, ibuf, crow, xrow, semg, sems):
    cid = lax.axis_index("c")
    sid = lax.axis_index("s")
    wid = sid * 2 + cid
    slot_base = wid * SLOTS_PER
    iota = lax.iota(jnp.int32, 16)

    pltpu.sync_copy(win_hbm.at[wid], wbuf)

    @pl.loop(0, SROWS)
    def _p2a(r):
        for t in range(SCOLS // 16):
            w = wbuf[r, pl.ds(t * 16, 16)]
            ls = r * SCOLS + t * 16 + iota
            dummy = N_QUAD + (ls & 2047)
            wbuf[r, pl.ds(t * 16, 16)] = jnp.where(w >= 0, w, dummy)

    @pl.loop(0, SROWS // 25)
    def _p2b(g):
        cps = []
        for b in range(25):
            c = g * 25 + b
            cps.append(pltpu.async_copy(abd_hbm.at[wbuf.at[c]],
                                        ibuf.at[c], semg))
        for cp in cps:
            cp.wait()

    @pl.loop(0, SROWS // 25)
    def _p2c(g):
        cps = []
        for b in range(25):
            c = g * 25 + b
            cps.append(pltpu.async_copy(db_hbm.at[ibuf.at[c]],
                                        wbuf.at[c], semg))
        for cp in cps:
            cp.wait()

    @pl.loop(0, SROWS // 5)
    def _p2d(g):
        gcps = []
        for b in range(5):
            c = g * 5 + b
            gcps.append(pltpu.async_copy(ccoef_hbm.at[ibuf.at[c]],
                                         crow.at[b], semg))
            gcps.append(pltpu.async_copy(xedge_hbm.at[wbuf.at[c]],
                                         xrow.at[b], semg))
        scps = []
        for b in range(5):
            c = g * 5 + b
            gcps[2 * b].wait()
            gcps[2 * b + 1].wait()
            for r in range(SCOLS):
                for t in range(2):
                    sl = pl.ds(t * 16, 16)
                    crow[b, r, sl] = crow[b, r, sl] * xrow[b, r, sl]
            base = slot_base + c * SCOLS
            scps.append(pltpu.async_copy(crow.at[b],
                                         m2x_hbm.at[pl.ds(base, SCOLS)], sems))
        for cp in scps:
            cp.wait()


def _sc_phase2(win, abd_pad, db_pad, ccoef_pad, x_edge):
    mesh = plsc.VectorSubcoreMesh(core_axis_name="c", subcore_axis_name="s")
    f = pl.kernel(
        _sc_p2_body,
        out_type=_SDS((NSLOT, 32), jnp.float32),
        mesh=mesh,
        compiler_params=_SC_PARAMS,
        scratch_types=[
            pltpu.VMEM((SROWS, SCOLS), jnp.int32),
            pltpu.VMEM((SROWS, SCOLS), jnp.int32),
            pltpu.VMEM((5, SCOLS, 32), jnp.float32),
            pltpu.VMEM((5, SCOLS, 32), jnp.float32),
            pltpu.SemaphoreType.DMA,
            pltpu.SemaphoreType.DMA,
        ],
    )
    return f(win, abd_pad, db_pad, ccoef_pad, x_edge)


def _sc_swap_body(x_hbm, idsw_hbm, out_hbm, idxbuf, rbuf, semg, sems):
    cid = lax.axis_index("c")
    sid = lax.axis_index("s")
    wid = sid * 2 + cid
    pltpu.sync_copy(idsw_hbm.at[wid], idxbuf)

    @pl.loop(0, 25)
    def _g(g):
        gcps = []
        for b in range(5):
            c = g * 5 + b
            gcps.append(pltpu.async_copy(x_hbm.at[idxbuf.at[c]],
                                         rbuf.at[b], semg))
        scps = []
        for b in range(5):
            c = g * 5 + b
            gcps[b].wait()
            base = wid * 5000 + c * 40
            scps.append(pltpu.async_copy(rbuf.at[b],
                                         out_hbm.at[pl.ds(base, 40)], sems))
        for cp in scps:
            cp.wait()


def _sc_swap(x, id_swap):
    mesh = plsc.VectorSubcoreMesh(core_axis_name="c", subcore_axis_name="s")
    f = pl.kernel(
        _sc_swap_body,
        out_type=_SDS((N_EDGES, 32), jnp.float32),
        mesh=mesh,
        compiler_params=_SC_PARAMS,
        scratch_types=[
            pltpu.VMEM((125, 40), jnp.int32),
            pltpu.VMEM((5, 40, 32), jnp.float32),
            pltpu.SemaphoreType.DMA,
            pltpu.SemaphoreType.DMA,
        ],
    )
    return f(x, id_swap.reshape(NW, 125, 40))


def kernel(m, rbf, cbf, sbf_rbfW1, sbf_sph, Kidx4, id_swap, id4_reduce_ca,
           id4_expand_intm_db, id4_expand_abd,
           W_db, W_rbf, W_cbf, W_down, W_bil, W_up_ca, W_up_ac):
    ca3 = id4_reduce_ca.reshape(QCH, QROWS, 16)
    kid3 = Kidx4.reshape(QCH, QROWS, 16)
    win = _sc_phase1(ca3, kid3)

    x_edge = _edge_dense(m, rbf, W_db, W_rbf, W_down)
    ccoef_pad = _matmul_pad(cbf, W_cbf, PAD)

    abd_pad = jnp.concatenate(
        [id4_expand_abd, N_INTM + jnp.arange(PAD, dtype=jnp.int32)])
    db_pad = jnp.concatenate(
        [id4_expand_intm_db,
         (jnp.arange(PAD, dtype=jnp.int32) * 19) % N_EDGES])

    m2 = _sc_phase2(win, abd_pad, db_pad, ccoef_pad, x_edge)

    rbfw1T = sbf_rbfW1.transpose(0, 2, 1).reshape(NSLOT, 32)
    W2 = W_bil.transpose(1, 0, 2).reshape(32 * 32, 32)
    x = _bilinear(sbf_sph.reshape(N_EDGES, 64),
                  rbfw1T, m2, W2)
    x_sw = _sc_swap(x, id_swap)
    return _up(x, x_sw, W_up_ca, W_up_ac)
